# Initial kernel scaffold; baseline (speedup 1.0000x reference)
import jax
import jax.numpy as jnp
from jax.experimental import pallas as pl


def kernel(x, Wg, expert_bias, W1, W2, W3, Ws1, Ws2, Ws3):
    raise NotImplementedError("write your pallas kernel here")


# speedup vs baseline: 1.8401x; 1.8401x over previous
import functools

import jax
import jax.numpy as jnp
from jax.experimental import pallas as pl
from jax.experimental.pallas import tpu as pltpu

DIM = 1024
E = 8
TOPK = 2
INTER = 512
T = 2048
BT = 256


def _silu(v):
    return v * jax.nn.sigmoid(v)


def _gating_body(x_ref, wg_ref, bias_ref, comb_ref):
    x = x_ref[...]
    wg = wg_ref[...]
    scores = jax.nn.sigmoid(
        jax.lax.dot_general(x, wg, (((1,), (1,)), ((), ())),
                            preferred_element_type=jnp.float32))
    biased = scores + bias_ref[...]
    lane = jax.lax.broadcasted_iota(jnp.int32, (BT, E), 1)
    m0 = jnp.max(biased, axis=1, keepdims=True)
    i0 = jnp.min(jnp.where(biased == m0, lane, E), axis=1, keepdims=True)
    masked = jnp.where(lane == i0, -jnp.inf, biased)
    m1 = jnp.max(masked, axis=1, keepdims=True)
    i1 = jnp.min(jnp.where(masked == m1, lane, E), axis=1, keepdims=True)
    w0 = jnp.sum(jnp.where(lane == i0, scores, 0.0), axis=1, keepdims=True)
    w1 = jnp.sum(jnp.where(lane == i1, scores, 0.0), axis=1, keepdims=True)
    comb_ref[...] = jnp.where(lane == i0, w0, jnp.where(lane == i1, w1, 0.0))


def _experts_body(x_ref, w1_ref, w3_ref, w2_ref, comb_ref, y_ref):
    x = x_ref[...]
    acc = jnp.zeros((BT, DIM), jnp.float32)
    comb = comb_ref[...]
    lane = jax.lax.broadcasted_iota(jnp.int32, (BT, E), 1)

    def body(e, acc):
        w1 = w1_ref[e]
        w3 = w3_ref[e]
        w2 = w2_ref[e]
        h = _silu(jax.lax.dot_general(x, w1, (((1,), (1,)), ((), ())),
                                      preferred_element_type=jnp.float32))
        h = h * jax.lax.dot_general(x, w3, (((1,), (1,)), ((), ())),
                                    preferred_element_type=jnp.float32)
        eo = jax.lax.dot_general(h, w2, (((1,), (1,)), ((), ())),
                                 preferred_element_type=jnp.float32)
        c = jnp.sum(jnp.where(lane == e, comb, 0.0), axis=1, keepdims=True)
        return acc + c * eo

    y_ref[...] = jax.lax.fori_loop(0, E, body, acc)


def _shared_body(x_ref, y_ref, ws1_ref, ws3_ref, ws2_ref, o_ref):
    x = x_ref[...]
    h = _silu(jax.lax.dot_general(x, ws1_ref[...], (((1,), (1,)), ((), ())),
                                  preferred_element_type=jnp.float32))
    h = h * jax.lax.dot_general(x, ws3_ref[...], (((1,), (1,)), ((), ())),
                                preferred_element_type=jnp.float32)
    z = jax.lax.dot_general(h, ws2_ref[...], (((1,), (1,)), ((), ())),
                            preferred_element_type=jnp.float32)
    o_ref[...] = z + y_ref[...]


@jax.jit
def _run(x, Wg, expert_bias, W1, W2, W3, Ws1, Ws2, Ws3):
    shape = x.shape
    xf = x.reshape(-1, DIM)
    bias2 = expert_bias.reshape(1, E)

    comb = pl.pallas_call(
        _gating_body,
        grid=(T // BT,),
        in_specs=[
            pl.BlockSpec((BT, DIM), lambda i: (i, 0)),
            pl.BlockSpec((E, DIM), lambda i: (0, 0)),
            pl.BlockSpec((1, E), lambda i: (0, 0)),
        ],
        out_specs=pl.BlockSpec((BT, E), lambda i: (i, 0)),
        out_shape=jax.ShapeDtypeStruct((T, E), jnp.float32),
    )(xf, Wg, bias2)

    y = pl.pallas_call(
        _experts_body,
        grid=(T // BT,),
        in_specs=[
            pl.BlockSpec((BT, DIM), lambda i: (i, 0)),
            pl.BlockSpec((E, INTER, DIM), lambda i: (0, 0, 0)),
            pl.BlockSpec((E, INTER, DIM), lambda i: (0, 0, 0)),
            pl.BlockSpec((E, DIM, INTER), lambda i: (0, 0, 0)),
            pl.BlockSpec((BT, E), lambda i: (i, 0)),
        ],
        out_specs=pl.BlockSpec((BT, DIM), lambda i: (i, 0)),
        out_shape=jax.ShapeDtypeStruct((T, DIM), jnp.float32),
    )(xf, W1, W3, W2, comb)

    out = pl.pallas_call(
        _shared_body,
        grid=(T // BT,),
        in_specs=[
            pl.BlockSpec((BT, DIM), lambda i: (i, 0)),
            pl.BlockSpec((BT, DIM), lambda i: (i, 0)),
            pl.BlockSpec((2 * INTER, DIM), lambda i: (0, 0)),
            pl.BlockSpec((2 * INTER, DIM), lambda i: (0, 0)),
            pl.BlockSpec((DIM, 2 * INTER), lambda i: (0, 0)),
        ],
        out_specs=pl.BlockSpec((BT, DIM), lambda i: (i, 0)),
        out_shape=jax.ShapeDtypeStruct((T, DIM), jnp.float32),
    )(xf, y, Ws1, Ws3, Ws2)

    return out.reshape(shape)


def kernel(x, Wg, expert_bias, W1, W2, W3, Ws1, Ws2, Ws3):
    return _run(x, Wg, expert_bias, W1, W2, W3, Ws1, Ws2, Ws3)
